# baseline (speedup 1.0000x reference)
import jax
import jax.numpy as jnp
from jax import lax
from jax.experimental import pallas as pl
from jax.experimental.pallas import tpu as pltpu
from jax.experimental.pallas import tpu_sc as plsc

B = 4
S = 8192
H = 768
SENT = 128
PAD = 1
EPS = 1e-05
NW = 32
TOK_PER_W = (B * S) // NW
CH = 64
NCH = TOK_PER_W // CH
POS_PAD = 136


def _make_gather_body(tok_per_w, nch):
  def _gather_body(ids_hbm, word_hbm, out_hbm, ids_v, rows_a, rows_b,
                   sga, sgb, swa, swb, sid):
    ci = lax.axis_index("c")
    si = lax.axis_index("s")
    w = si * 2 + ci
    base = w * tok_per_w
    pltpu.async_copy(ids_hbm.at[pl.ds(base, tok_per_w)], ids_v, sid).wait()

    def fire_gather(i, buf, sem):
        pltpu.async_copy(word_hbm.at[ids_v.at[pl.ds(i * CH, CH)]], buf, sem)

    def wait_gather(i, buf, sem):
        pltpu.make_async_copy(word_hbm.at[ids_v.at[pl.ds(i * CH, CH)]],
                              buf, sem).wait()

    def fire_write(i, buf, sem):
        pltpu.async_copy(buf, out_hbm.at[pl.ds(base + i * CH, CH)], sem)

    def wait_write(i, buf, sem):
        pltpu.make_async_copy(buf, out_hbm.at[pl.ds(base + i * CH, CH)],
                              sem).wait()

    fire_gather(0, rows_a, sga)

    def step(m, _):
        i0 = 2 * m
        i1 = 2 * m + 1

        @pl.when(m > 0)
        def _():
            wait_write(i1 - 2, rows_b, swb)

        fire_gather(i1, rows_b, sgb)
        wait_gather(i0, rows_a, sga)
        fire_write(i0, rows_a, swa)

        @pl.when(m < nch // 2 - 1)
        def _():
            wait_write(i0, rows_a, swa)
            fire_gather(i0 + 2, rows_a, sga)

        wait_gather(i1, rows_b, sgb)
        fire_write(i1, rows_b, swb)
        return 0

    lax.fori_loop(0, nch // 2, step, 0)
    wait_write(nch - 2, rows_a, swa)
    wait_write(nch - 1, rows_b, swb)

  return _gather_body


def _sc_gather(ids_flat, word_emb):
    ntok = ids_flat.shape[0]
    tok_per_w = ntok // NW
    nch = tok_per_w // CH
    mesh = plsc.VectorSubcoreMesh(core_axis_name="c", subcore_axis_name="s")
    f = pl.kernel(
        _make_gather_body(tok_per_w, nch),
        out_type=jax.ShapeDtypeStruct((ntok, H), jnp.float32),
        mesh=mesh,
        compiler_params=pltpu.CompilerParams(needs_layout_passes=False),
        scratch_types=[
            pltpu.VMEM((tok_per_w,), jnp.int32),
            pltpu.VMEM((CH, H), jnp.float32),
            pltpu.VMEM((CH, H), jnp.float32),
            pltpu.SemaphoreType.DMA,
            pltpu.SemaphoreType.DMA,
            pltpu.SemaphoreType.DMA,
            pltpu.SemaphoreType.DMA,
            pltpu.SemaphoreType.DMA,
        ],
    )
    return f(ids_flat, word_emb)


BLKR = 1024


def _ln_body(prev_ref, x_ref, ids_ref, cs_ref, c0_ref, tt_ref, g_ref, b_ref,
             o_ref):
    del prev_ref
    x = x_ref[...]
    idc = ids_ref[0]
    tt = tt_ref[...]
    cs = cs_ref[...] + tt
    c0 = c0_ref[...] + tt
    mf = (idc != PAD).astype(jnp.float32)
    x = x + c0 + mf * (cs - c0)
    ones = jnp.ones((H, 1), jnp.float32)
    dn = (((1,), (0,)), ((), ()))
    s1 = lax.dot_general(x, ones, dn, preferred_element_type=jnp.float32)
    s2 = lax.dot_general(x * x, ones, dn, preferred_element_type=jnp.float32)
    mu = s1 * (1.0 / H)
    var = s2 * (1.0 / H) - mu * mu
    y = (x - mu) * lax.rsqrt(var + EPS)
    o_ref[...] = y * g_ref[...] + b_ref[...]


def _tc_ln_half(prev, gathered, ids3, cs, c0, tt1, g1, b1, base_blk):
    nblk = gathered.shape[0] // BLKR
    return pl.pallas_call(
        _ln_body,
        grid=(nblk,),
        in_specs=[
            pl.BlockSpec(memory_space=pltpu.MemorySpace.HBM),
            pl.BlockSpec((BLKR, H), lambda i: (i, 0)),
            pl.BlockSpec((1, BLKR, 1), lambda i: (i, 0, 0)),
            pl.BlockSpec((BLKR, H), lambda i: (0, 0)),
            pl.BlockSpec((1, H), lambda i: (0, 0)),
            pl.BlockSpec((1, H), lambda i: (0, 0)),
            pl.BlockSpec((1, H), lambda i: (0, 0)),
            pl.BlockSpec((1, H), lambda i: (0, 0)),
        ],
        out_specs=pl.BlockSpec((BLKR, H), lambda i: (i + base_blk, 0)),
        out_shape=jax.ShapeDtypeStruct((B * S, H), jnp.float32),
        input_output_aliases={0: 0},
    )(prev, gathered, ids3, cs, c0, tt1, g1, b1)


@jax.jit
def _run(input_ids, word_emb, pos_emb, tt_emb, gamma, beta):
    n2 = (B * S) // 2
    ids_flat = input_ids.reshape(B * S)
    gath1 = _sc_gather(ids_flat[:n2], word_emb)
    gath2 = _sc_gather(ids_flat[n2:], word_emb)
    ids3 = input_ids.reshape((B * S) // BLKR, BLKR, 1)
    nb2 = (B * S) // BLKR // 2
    cs = jnp.tile(pos_emb[2:SENT + 2], (BLKR // SENT, 1))
    c0 = pos_emb[0:1]
    tt1 = tt_emb[0:1]
    g1 = gamma.reshape(1, H)
    b1 = beta.reshape(1, H)
    buf = jax.lax.full((B * S, H), 0.0, jnp.float32)
    o1 = _tc_ln_half(buf, gath1, ids3[:nb2], cs, c0, tt1, g1, b1, 0)
    out = _tc_ln_half(o1, gath2, ids3[nb2:], cs, c0, tt1, g1, b1, nb2)
    return out.reshape(B, S, H)


def kernel(input_ids, word_emb, pos_emb, tt_emb, gamma, beta, position_ids_buf):
    del position_ids_buf
    return _run(input_ids, word_emb, pos_emb, tt_emb, gamma, beta)


# speedup vs baseline: 1.1678x; 1.1678x over previous
import jax
import jax.numpy as jnp
from jax import lax
from jax.experimental import pallas as pl
from jax.experimental.pallas import tpu as pltpu
from jax.experimental.pallas import tpu_sc as plsc

B = 4
S = 8192
H = 768
SENT = 128
PAD = 1
EPS = 1e-05
NW = 32
TOK_PER_W = (B * S) // NW
CH = 64
NCH = TOK_PER_W // CH
POS_PAD = 136


def _make_gather_body(tok_per_w, nch):
  def _gather_body(ids_hbm, word_hbm, out_hbm, ids_v, rows_a, rows_b,
                   sga, sgb, swa, swb, sid):
    ci = lax.axis_index("c")
    si = lax.axis_index("s")
    w = si * 2 + ci
    base = w * tok_per_w
    pltpu.async_copy(ids_hbm.at[pl.ds(base, tok_per_w)], ids_v, sid).wait()

    def fire_gather(i, buf, sem):
        pltpu.async_copy(word_hbm.at[ids_v.at[pl.ds(i * CH, CH)]], buf, sem)

    def wait_gather(i, buf, sem):
        pltpu.make_async_copy(word_hbm.at[ids_v.at[pl.ds(i * CH, CH)]],
                              buf, sem).wait()

    def fire_write(i, buf, sem):
        pltpu.async_copy(buf, out_hbm.at[pl.ds(base + i * CH, CH)], sem)

    def wait_write(i, buf, sem):
        pltpu.make_async_copy(buf, out_hbm.at[pl.ds(base + i * CH, CH)],
                              sem).wait()

    fire_gather(0, rows_a, sga)

    def step(m, _):
        i0 = 2 * m
        i1 = 2 * m + 1

        @pl.when(m > 0)
        def _():
            wait_write(i1 - 2, rows_b, swb)

        fire_gather(i1, rows_b, sgb)
        wait_gather(i0, rows_a, sga)
        fire_write(i0, rows_a, swa)

        @pl.when(m < nch // 2 - 1)
        def _():
            wait_write(i0, rows_a, swa)
            fire_gather(i0 + 2, rows_a, sga)

        wait_gather(i1, rows_b, sgb)
        fire_write(i1, rows_b, swb)
        return 0

    lax.fori_loop(0, nch // 2, step, 0)
    wait_write(nch - 2, rows_a, swa)
    wait_write(nch - 1, rows_b, swb)

  return _gather_body


def _sc_gather(ids_flat, word_emb):
    ntok = ids_flat.shape[0]
    tok_per_w = ntok // NW
    nch = tok_per_w // CH
    mesh = plsc.VectorSubcoreMesh(core_axis_name="c", subcore_axis_name="s")
    f = pl.kernel(
        _make_gather_body(tok_per_w, nch),
        out_type=jax.ShapeDtypeStruct((ntok, H), jnp.float32),
        mesh=mesh,
        compiler_params=pltpu.CompilerParams(needs_layout_passes=False),
        scratch_types=[
            pltpu.VMEM((tok_per_w,), jnp.int32),
            pltpu.VMEM((CH, H), jnp.float32),
            pltpu.VMEM((CH, H), jnp.float32),
            pltpu.SemaphoreType.DMA,
            pltpu.SemaphoreType.DMA,
            pltpu.SemaphoreType.DMA,
            pltpu.SemaphoreType.DMA,
            pltpu.SemaphoreType.DMA,
        ],
    )
    return f(ids_flat, word_emb)


BLKR = 1024


def _ln_body_first(x_ref, ids_ref, cs_ref, c0_ref, tt_ref, g_ref, b_ref,
                   o_ref):
    _ln_math(x_ref, ids_ref, cs_ref, c0_ref, tt_ref, g_ref, b_ref, o_ref)


def _ln_body(prev_ref, x_ref, ids_ref, cs_ref, c0_ref, tt_ref, g_ref, b_ref,
             o_ref):
    del prev_ref
    _ln_math(x_ref, ids_ref, cs_ref, c0_ref, tt_ref, g_ref, b_ref, o_ref)


def _ln_math(x_ref, ids_ref, cs_ref, c0_ref, tt_ref, g_ref, b_ref, o_ref):
    x = x_ref[...]
    idc = ids_ref[0]
    tt = tt_ref[...]
    cs = cs_ref[...] + tt
    c0 = c0_ref[...] + tt
    mf = (idc != PAD).astype(jnp.float32)
    x = x + c0 + mf * (cs - c0)
    ones = jnp.ones((H, 1), jnp.float32)
    dn = (((1,), (0,)), ((), ()))
    s1 = lax.dot_general(x, ones, dn, preferred_element_type=jnp.float32)
    s2 = lax.dot_general(x * x, ones, dn, preferred_element_type=jnp.float32)
    mu = s1 * (1.0 / H)
    var = s2 * (1.0 / H) - mu * mu
    y = (x - mu) * lax.rsqrt(var + EPS)
    o_ref[...] = y * g_ref[...] + b_ref[...]


def _tc_ln_half(prev, gathered, ids3, cs, c0, tt1, g1, b1, base_blk):
    nblk = gathered.shape[0] // BLKR
    common = dict(
        grid=(nblk,),
        out_shape=jax.ShapeDtypeStruct((B * S, H), jnp.float32),
    )
    data_specs = [
        pl.BlockSpec((BLKR, H), lambda i: (i, 0)),
        pl.BlockSpec((1, BLKR, 1), lambda i: (i, 0, 0)),
        pl.BlockSpec((BLKR, H), lambda i: (0, 0)),
        pl.BlockSpec((1, H), lambda i: (0, 0)),
        pl.BlockSpec((1, H), lambda i: (0, 0)),
        pl.BlockSpec((1, H), lambda i: (0, 0)),
        pl.BlockSpec((1, H), lambda i: (0, 0)),
    ]
    out_spec = pl.BlockSpec((BLKR, H), lambda i: (i + base_blk, 0))
    if prev is None:
        return pl.pallas_call(
            _ln_body_first, in_specs=data_specs, out_specs=out_spec, **common,
        )(gathered, ids3, cs, c0, tt1, g1, b1)
    return pl.pallas_call(
        _ln_body,
        in_specs=[pl.BlockSpec(memory_space=pltpu.MemorySpace.HBM)] + data_specs,
        out_specs=out_spec,
        input_output_aliases={0: 0},
        **common,
    )(prev, gathered, ids3, cs, c0, tt1, g1, b1)


@jax.jit
def _run(input_ids, word_emb, pos_emb, tt_emb, gamma, beta):
    n2 = (B * S) // 2
    ids_flat = input_ids.reshape(B * S)
    gath1 = _sc_gather(ids_flat[:n2], word_emb)
    gath2 = _sc_gather(ids_flat[n2:], word_emb)
    ids3 = input_ids.reshape((B * S) // BLKR, BLKR, 1)
    nb2 = (B * S) // BLKR // 2
    cs = jnp.tile(pos_emb[2:SENT + 2], (BLKR // SENT, 1))
    c0 = pos_emb[0:1]
    tt1 = tt_emb[0:1]
    g1 = gamma.reshape(1, H)
    b1 = beta.reshape(1, H)
    o1 = _tc_ln_half(None, gath1, ids3[:nb2], cs, c0, tt1, g1, b1, 0)
    out = _tc_ln_half(o1, gath2, ids3[nb2:], cs, c0, tt1, g1, b1, nb2)
    return out.reshape(B, S, H)


def kernel(input_ids, word_emb, pos_emb, tt_emb, gamma, beta, position_ids_buf):
    del position_ids_buf
    return _run(input_ids, word_emb, pos_emb, tt_emb, gamma, beta)


# speedup vs baseline: 1.2468x; 1.0677x over previous
import jax
import jax.numpy as jnp
from jax import lax
from jax.experimental import pallas as pl
from jax.experimental.pallas import tpu as pltpu
from jax.experimental.pallas import tpu_sc as plsc

B = 4
S = 8192
H = 768
SENT = 128
PAD = 1
EPS = 1e-05
NW = 32
TOK_PER_W = (B * S) // NW
CH = 64
NCH = TOK_PER_W // CH
POS_PAD = 136


def _gather_body(ids_hbm, word_hbm, out_hbm, ids_v, rows_a, rows_b,
                 sga, sgb, swa, swb, sid):
    ci = lax.axis_index("c")
    si = lax.axis_index("s")
    w = si * 2 + ci
    base = w * TOK_PER_W
    pltpu.async_copy(ids_hbm.at[pl.ds(base, TOK_PER_W)], ids_v, sid).wait()

    def fire_gather(i, buf, sem):
        pltpu.async_copy(word_hbm.at[ids_v.at[pl.ds(i * CH, CH)]], buf, sem)

    def wait_gather(i, buf, sem):
        pltpu.make_async_copy(word_hbm.at[ids_v.at[pl.ds(i * CH, CH)]],
                              buf, sem).wait()

    def fire_write(i, buf, sem):
        pltpu.async_copy(buf, out_hbm.at[pl.ds(base + i * CH, CH)], sem)

    def wait_write(i, buf, sem):
        pltpu.make_async_copy(buf, out_hbm.at[pl.ds(base + i * CH, CH)],
                              sem).wait()

    fire_gather(0, rows_a, sga)

    def step(m, _):
        i0 = 2 * m
        i1 = 2 * m + 1

        @pl.when(m > 0)
        def _():
            wait_write(i1 - 2, rows_b, swb)

        fire_gather(i1, rows_b, sgb)
        wait_gather(i0, rows_a, sga)
        fire_write(i0, rows_a, swa)

        @pl.when(m < NCH // 2 - 1)
        def _():
            wait_write(i0, rows_a, swa)
            fire_gather(i0 + 2, rows_a, sga)

        wait_gather(i1, rows_b, sgb)
        fire_write(i1, rows_b, swb)
        return 0

    lax.fori_loop(0, NCH // 2, step, 0)
    wait_write(NCH - 2, rows_a, swa)
    wait_write(NCH - 1, rows_b, swb)


def _sc_gather(ids_flat, word_emb):
    mesh = plsc.VectorSubcoreMesh(core_axis_name="c", subcore_axis_name="s")
    f = pl.kernel(
        _gather_body,
        out_type=jax.ShapeDtypeStruct((B * S, H), jnp.float32),
        mesh=mesh,
        compiler_params=pltpu.CompilerParams(needs_layout_passes=False),
        scratch_types=[
            pltpu.VMEM((TOK_PER_W,), jnp.int32),
            pltpu.VMEM((CH, H), jnp.float32),
            pltpu.VMEM((CH, H), jnp.float32),
            pltpu.SemaphoreType.DMA,
            pltpu.SemaphoreType.DMA,
            pltpu.SemaphoreType.DMA,
            pltpu.SemaphoreType.DMA,
            pltpu.SemaphoreType.DMA,
        ],
    )
    return f(ids_flat, word_emb)


BLKR = 1024


def _ln_body(x_ref, ids_ref, cs_ref, c0_ref, tt_ref, g_ref, b_ref, o_ref,
             delta_ref, c0t_ref):
    @pl.when(pl.program_id(0) == 0)
    def _():
        delta_ref[...] = cs_ref[...] - c0_ref[...]
        c0t_ref[...] = c0_ref[...] + tt_ref[...]

    x = x_ref[...]
    idc = ids_ref[0]
    mf = (idc != PAD).astype(jnp.float32)
    x = x + c0t_ref[...] + mf * delta_ref[...]
    ones = jnp.ones((H, 1), jnp.float32)
    dn = (((1,), (0,)), ((), ()))
    s1 = lax.dot_general(x, ones, dn, preferred_element_type=jnp.float32)
    s2 = lax.dot_general(x * x, ones, dn, preferred_element_type=jnp.float32)
    mu = s1 * (1.0 / H)
    var = s2 * (1.0 / H) - mu * mu
    y = (x - mu) * lax.rsqrt(var + EPS)
    o_ref[...] = y * g_ref[...] + b_ref[...]


def _tc_ln(gathered, ids3, cs, c0, tt1, g1, b1):
    nblk = (B * S) // BLKR
    return pl.pallas_call(
        _ln_body,
        grid=(nblk,),
        in_specs=[
            pl.BlockSpec((BLKR, H), lambda i: (i, 0)),
            pl.BlockSpec((1, BLKR, 1), lambda i: (i, 0, 0)),
            pl.BlockSpec((BLKR, H), lambda i: (0, 0)),
            pl.BlockSpec((1, H), lambda i: (0, 0)),
            pl.BlockSpec((1, H), lambda i: (0, 0)),
            pl.BlockSpec((1, H), lambda i: (0, 0)),
            pl.BlockSpec((1, H), lambda i: (0, 0)),
        ],
        out_specs=pl.BlockSpec((BLKR, H), lambda i: (i, 0)),
        out_shape=jax.ShapeDtypeStruct((B * S, H), jnp.float32),
        scratch_shapes=[
            pltpu.VMEM((BLKR, H), jnp.float32),
            pltpu.VMEM((1, H), jnp.float32),
        ],
    )(gathered, ids3, cs, c0, tt1, g1, b1)


@jax.jit
def _run(input_ids, word_emb, pos_emb, tt_emb, gamma, beta):
    ids_flat = input_ids.reshape(B * S)
    gathered = _sc_gather(ids_flat, word_emb)
    ids3 = input_ids.reshape((B * S) // BLKR, BLKR, 1)
    cs = jnp.tile(pos_emb[2:SENT + 2], (BLKR // SENT, 1))
    c0 = pos_emb[0:1]
    tt1 = tt_emb[0:1]
    g1 = gamma.reshape(1, H)
    b1 = beta.reshape(1, H)
    out = _tc_ln(gathered, ids3, cs, c0, tt1, g1, b1)
    return out.reshape(B, S, H)


def kernel(input_ids, word_emb, pos_emb, tt_emb, gamma, beta, position_ids_buf):
    del position_ids_buf
    return _run(input_ids, word_emb, pos_emb, tt_emb, gamma, beta)


# speedup vs baseline: 1.2852x; 1.0308x over previous
import jax
import jax.numpy as jnp
from jax import lax
from jax.experimental import pallas as pl
from jax.experimental.pallas import tpu as pltpu
from jax.experimental.pallas import tpu_sc as plsc

B = 4
S = 8192
H = 768
SENT = 128
PAD = 1
EPS = 1e-05
NW = 32
TOK_PER_W = (B * S) // NW
CH = 64
NCH = TOK_PER_W // CH
POS_PAD = 136


def _gather_body(ids_hbm, word_hbm, out_hbm, ids_v, rows_a, rows_b,
                 sga, sgb, swa, swb, sid):
    ci = lax.axis_index("c")
    si = lax.axis_index("s")
    w = si * 2 + ci
    base = w * TOK_PER_W
    pltpu.async_copy(ids_hbm.at[pl.ds(base, TOK_PER_W)], ids_v, sid).wait()

    def fire_gather(i, buf, sem):
        pltpu.async_copy(word_hbm.at[ids_v.at[pl.ds(i * CH, CH)]], buf, sem)

    def wait_gather(i, buf, sem):
        pltpu.make_async_copy(word_hbm.at[ids_v.at[pl.ds(i * CH, CH)]],
                              buf, sem).wait()

    def fire_write(i, buf, sem):
        pltpu.async_copy(buf, out_hbm.at[pl.ds(base + i * CH, CH)], sem)

    def wait_write(i, buf, sem):
        pltpu.make_async_copy(buf, out_hbm.at[pl.ds(base + i * CH, CH)],
                              sem).wait()

    fire_gather(0, rows_a, sga)

    def step(m, _):
        i0 = 2 * m
        i1 = 2 * m + 1

        @pl.when(m > 0)
        def _():
            wait_write(i1 - 2, rows_b, swb)

        fire_gather(i1, rows_b, sgb)
        wait_gather(i0, rows_a, sga)
        fire_write(i0, rows_a, swa)

        @pl.when(m < NCH // 2 - 1)
        def _():
            wait_write(i0, rows_a, swa)
            fire_gather(i0 + 2, rows_a, sga)

        wait_gather(i1, rows_b, sgb)
        fire_write(i1, rows_b, swb)
        return 0

    lax.fori_loop(0, NCH // 2, step, 0)
    wait_write(NCH - 2, rows_a, swa)
    wait_write(NCH - 1, rows_b, swb)


def _sc_gather(ids_flat, word_emb):
    mesh = plsc.VectorSubcoreMesh(core_axis_name="c", subcore_axis_name="s")
    f = pl.kernel(
        _gather_body,
        out_type=jax.ShapeDtypeStruct((B * S, H), jnp.float32),
        mesh=mesh,
        compiler_params=pltpu.CompilerParams(needs_layout_passes=False),
        scratch_types=[
            pltpu.VMEM((TOK_PER_W,), jnp.int32),
            pltpu.VMEM((CH, H), jnp.float32),
            pltpu.VMEM((CH, H), jnp.float32),
            pltpu.SemaphoreType.DMA,
            pltpu.SemaphoreType.DMA,
            pltpu.SemaphoreType.DMA,
            pltpu.SemaphoreType.DMA,
            pltpu.SemaphoreType.DMA,
        ],
    )
    return f(ids_flat, word_emb)


BLKR = 2048


def _ln_body(x_ref, ids_ref, cs_ref, c0_ref, tt_ref, g_ref, b_ref, o_ref,
             delta_ref, c0t_ref):
    @pl.when(pl.program_id(0) == 0)
    def _():
        delta_ref[...] = cs_ref[...] - c0_ref[...]
        c0t_ref[...] = c0_ref[...] + tt_ref[...]

    x = x_ref[...]
    idc = ids_ref[0]
    mf = (idc != PAD).astype(jnp.float32)
    x = x + c0t_ref[...] + mf * delta_ref[...]
    ones = jnp.ones((H, 1), jnp.float32)
    dn = (((1,), (0,)), ((), ()))
    s1 = lax.dot_general(x, ones, dn, preferred_element_type=jnp.float32)
    s2 = lax.dot_general(x * x, ones, dn, preferred_element_type=jnp.float32)
    mu = s1 * (1.0 / H)
    var = s2 * (1.0 / H) - mu * mu
    y = (x - mu) * lax.rsqrt(var + EPS)
    o_ref[...] = y * g_ref[...] + b_ref[...]


def _tc_ln(gathered, ids3, cs, c0, tt1, g1, b1):
    nblk = (B * S) // BLKR
    return pl.pallas_call(
        _ln_body,
        grid=(nblk,),
        in_specs=[
            pl.BlockSpec((BLKR, H), lambda i: (i, 0)),
            pl.BlockSpec((1, BLKR, 1), lambda i: (i, 0, 0)),
            pl.BlockSpec((BLKR, H), lambda i: (0, 0)),
            pl.BlockSpec((1, H), lambda i: (0, 0)),
            pl.BlockSpec((1, H), lambda i: (0, 0)),
            pl.BlockSpec((1, H), lambda i: (0, 0)),
            pl.BlockSpec((1, H), lambda i: (0, 0)),
        ],
        out_specs=pl.BlockSpec((BLKR, H), lambda i: (i, 0)),
        out_shape=jax.ShapeDtypeStruct((B * S, H), jnp.float32),
        scratch_shapes=[
            pltpu.VMEM((BLKR, H), jnp.float32),
            pltpu.VMEM((1, H), jnp.float32),
        ],
    )(gathered, ids3, cs, c0, tt1, g1, b1)


@jax.jit
def _run(input_ids, word_emb, pos_emb, tt_emb, gamma, beta):
    ids_flat = input_ids.reshape(B * S)
    gathered = _sc_gather(ids_flat, word_emb)
    ids3 = input_ids.reshape((B * S) // BLKR, BLKR, 1)
    cs = jnp.tile(pos_emb[2:SENT + 2], (BLKR // SENT, 1))
    c0 = pos_emb[0:1]
    tt1 = tt_emb[0:1]
    g1 = gamma.reshape(1, H)
    b1 = beta.reshape(1, H)
    out = _tc_ln(gathered, ids3, cs, c0, tt1, g1, b1)
    return out.reshape(B, S, H)


def kernel(input_ids, word_emb, pos_emb, tt_emb, gamma, beta, position_ids_buf):
    del position_ids_buf
    return _run(input_ids, word_emb, pos_emb, tt_emb, gamma, beta)
